# baseline (speedup 1.0000x reference)
import functools

import jax
import jax.numpy as jnp
import numpy as np
from jax.experimental import pallas as pl
from jax.experimental.pallas import tpu as pltpu
from jax.sharding import Mesh, PartitionSpec as P


def _round_up(a, m):
    return (a + m - 1) // m * m


_TRANS_B = (((1,), (1,)), ((), ()))


def _fused_mlp_kernel(x_ref, w1_ref, b1_ref, w2_ref, b2_ref, o_ref):
    xb = x_ref[...].astype(jnp.bfloat16)
    h = jax.lax.dot_general(xb, w1_ref[...], _TRANS_B,
                            preferred_element_type=jnp.float32)
    h = h + b1_ref[...]
    g = 0.5 * h * (1.0 + jax.lax.erf(h * jnp.float32(0.7071067811865476)))
    o = jax.lax.dot_general(g.astype(jnp.bfloat16), w2_ref[...], _TRANS_B,
                            preferred_element_type=jnp.float32)
    o_ref[...] = o + b2_ref[...]


def _mlp_one_core(x, w1b, b1r, w2b, b2r, *, tm):
    n_tokens, in_features = x.shape
    hidden = w1b.shape[0]
    out_features = w2b.shape[0]

    tm_eff = max(128, min(_round_up(tm, 128), _round_up(n_tokens, 128)))
    n_pad = _round_up(n_tokens, tm_eff)
    if n_pad != n_tokens:
        x = jnp.pad(x, ((0, n_pad - n_tokens), (0, 0)))
    grid_len = n_pad // tm_eff

    flops = 2 * n_pad * (in_features * hidden + hidden * out_features)
    bytes_accessed = 4 * n_pad * (in_features + out_features) + 2 * (
        in_features * hidden + hidden * out_features) + 4 * (hidden + out_features)
    cost = pl.CostEstimate(flops=flops,
                           transcendentals=n_pad * hidden,
                           bytes_accessed=bytes_accessed)

    out = pl.pallas_call(
        _fused_mlp_kernel,
        out_shape=jax.ShapeDtypeStruct((n_pad, out_features), x.dtype),
        grid=(grid_len,),
        in_specs=[
            pl.BlockSpec((tm_eff, in_features), lambda i: (i, 0)),
            pl.BlockSpec((hidden, in_features), lambda i: (0, 0)),
            pl.BlockSpec((1, hidden), lambda i: (0, 0)),
            pl.BlockSpec((out_features, hidden), lambda i: (0, 0)),
            pl.BlockSpec((1, out_features), lambda i: (0, 0)),
        ],
        out_specs=pl.BlockSpec((tm_eff, out_features), lambda i: (i, 0)),
        compiler_params=pltpu.CompilerParams(
            dimension_semantics=("parallel",),
            vmem_limit_bytes=64 << 20),
        cost_estimate=cost,
    )(x, w1b, b1r, w2b, b2r)

    return out[:n_tokens]


def kernel(x, w1, b1, w2, b2, *, tm=1024):
    in_features = x.shape[-1]
    hidden = w1.shape[0]
    out_features = w2.shape[0]
    lead = x.shape[:-1]

    x2 = x.reshape(-1, in_features)
    n_tokens = x2.shape[0]

    w1b = w1.astype(jnp.bfloat16)
    w2b = w2.astype(jnp.bfloat16)
    b1r = b1.reshape(1, hidden)
    b2r = b2.reshape(1, out_features)

    one_core = functools.partial(_mlp_one_core, tm=tm)

    devs = jax.devices()
    if len(devs) >= 2 and n_tokens % 2 == 0:
        mesh = Mesh(np.array(devs[:2]), ("tc",))
        fn = jax.shard_map(one_core, mesh=mesh,
                           in_specs=(P("tc"), P(), P(), P(), P()),
                           out_specs=P("tc"), check_vma=False)
        out = fn(x2, w1b, b1r, w2b, b2r)
    else:
        out = one_core(x2, w1b, b1r, w2b, b2r)

    return out.reshape(*lead, out_features)


# speedup vs baseline: 6.4924x; 6.4924x over previous
import jax
import jax.numpy as jnp
from jax.experimental import pallas as pl
from jax.experimental.pallas import tpu as pltpu


def _round_up(a, m):
    return (a + m - 1) // m * m


_TRANS_B = (((1,), (1,)), ((), ()))


def _fused_mlp_kernel(x_ref, w1_ref, b1_ref, w2_ref, b2_ref, o_ref,
                      w1s_ref, w2s_ref):
    @pl.when(pl.program_id(0) == 0)
    def _cast_weights_once():
        w1s_ref[...] = w1_ref[...].astype(jnp.bfloat16)
        w2s_ref[...] = w2_ref[...].astype(jnp.bfloat16)

    xb = x_ref[...].astype(jnp.bfloat16)
    h = jax.lax.dot_general(xb, w1s_ref[...], _TRANS_B,
                            preferred_element_type=jnp.float32)
    h = h + b1_ref[...]
    g = 0.5 * h * (1.0 + jax.lax.erf(h * jnp.float32(0.7071067811865476)))
    o = jax.lax.dot_general(g.astype(jnp.bfloat16), w2s_ref[...], _TRANS_B,
                            preferred_element_type=jnp.float32)
    o_ref[...] = o + b2_ref[...]


def kernel(x, w1, b1, w2, b2, *, tm=1024):
    in_features = x.shape[-1]
    hidden = w1.shape[0]
    out_features = w2.shape[0]
    lead = x.shape[:-1]

    x2 = x.reshape(-1, in_features)
    n_tokens = x2.shape[0]

    tm_eff = max(128, min(_round_up(tm, 128), _round_up(n_tokens, 128)))
    n_pad = _round_up(n_tokens, tm_eff)
    if n_pad != n_tokens:
        x2 = jnp.pad(x2, ((0, n_pad - n_tokens), (0, 0)))
    grid_len = n_pad // tm_eff

    b1r = b1.reshape(1, hidden)
    b2r = b2.reshape(1, out_features)

    flops = 2 * n_pad * (in_features * hidden + hidden * out_features)
    bytes_accessed = 4 * n_pad * (in_features + out_features) + 4 * (
        in_features * hidden + hidden * out_features) + 4 * (hidden + out_features)
    cost = pl.CostEstimate(flops=flops,
                           transcendentals=n_pad * hidden,
                           bytes_accessed=bytes_accessed)

    out = pl.pallas_call(
        _fused_mlp_kernel,
        out_shape=jax.ShapeDtypeStruct((n_pad, out_features), x.dtype),
        grid=(grid_len,),
        in_specs=[
            pl.BlockSpec((tm_eff, in_features), lambda i: (i, 0)),
            pl.BlockSpec((hidden, in_features), lambda i: (0, 0)),
            pl.BlockSpec((1, hidden), lambda i: (0, 0)),
            pl.BlockSpec((out_features, hidden), lambda i: (0, 0)),
            pl.BlockSpec((1, out_features), lambda i: (0, 0)),
        ],
        out_specs=pl.BlockSpec((tm_eff, out_features), lambda i: (i, 0)),
        scratch_shapes=[
            pltpu.VMEM((hidden, in_features), jnp.bfloat16),
            pltpu.VMEM((out_features, hidden), jnp.bfloat16),
        ],
        compiler_params=pltpu.CompilerParams(
            dimension_semantics=("arbitrary",),
            vmem_limit_bytes=64 << 20),
        cost_estimate=cost,
    )(x2, w1, b1r, w2, b2r)

    out = out[:n_tokens]
    return out.reshape(*lead, out_features)
